# Initial kernel scaffold; baseline (speedup 1.0000x reference)
import jax
import jax.numpy as jnp
import numpy as np
from jax import lax
from jax.experimental import pallas as pl
from jax.experimental.pallas import tpu as pltpu
from jax.experimental.pallas import tpu_sc as plsc

N = 10000
E = 320000
FIN = 128
H = 4
C = 32
HC = H * C
NGRAPH = 64
AUG = 144

NCORES = 2
NSUB = 16
NW = NCORES * NSUB
EPT = E // NW
G = 80
GPT = EPT // G
NPC = N // NSUB

_f32 = jnp.float32


def _proj_kernel(x_ref, w1_ref, p_ref, xh_ref, asd_ref):
  xh = jnp.dot(x_ref[...], w1_ref[...], preferred_element_type=_f32)
  xh_ref[...] = xh
  asd_ref[...] = jnp.dot(xh, p_ref[...], preferred_element_type=_f32)


def _run_proj(x, W1, P):
  bn = 1000
  return pl.pallas_call(
      _proj_kernel,
      grid=(N // bn,),
      in_specs=[
          pl.BlockSpec((bn, FIN), lambda i: (i, 0)),
          pl.BlockSpec((FIN, HC), lambda i: (0, 0)),
          pl.BlockSpec((HC, 2 * H), lambda i: (0, 0)),
      ],
      out_specs=[
          pl.BlockSpec((bn, HC), lambda i: (i, 0)),
          pl.BlockSpec((bn, 2 * H), lambda i: (i, 0)),
      ],
      out_shape=[
          jax.ShapeDtypeStruct((N, HC), _f32),
          jax.ShapeDtypeStruct((N, 2 * H), _f32),
      ],
  )(x, W1, P)


def _edge_kernel(srcf_hbm, dst2d_hbm, asd_hbm, xh_hbm, zeros_hbm, parts_hbm,
                 asd_v, srcf_v, dst2d_v, dstrow, rows, stag, agg_sh, sem):
  c = lax.axis_index("c")
  s = lax.axis_index("s")
  wid = c * NSUB + s

  pltpu.sync_copy(asd_hbm, asd_v)
  pltpu.sync_copy(srcf_hbm.at[pl.ds(wid * EPT, EPT)], srcf_v)
  pltpu.sync_copy(dst2d_hbm.at[pl.ds(wid * GPT, GPT)], dst2d_v)
  pltpu.sync_copy(zeros_hbm, agg_sh.at[pl.ds(s * NPC, NPC)])
  plsc.subcore_barrier()

  iota16 = lax.iota(jnp.int32, 16)
  head_masks = [iota16 == h for h in range(H)]
  zero16 = jnp.zeros((16,), _f32)

  def group(j, carry):
    gat = pltpu.async_copy(xh_hbm.at[srcf_v.at[pl.ds(j * G, G)]], rows, sem)
    pltpu.sync_copy(dst2d_v.at[j], dstrow)

    exs = []
    for t in range(G // 16):
      src16 = srcf_v[pl.ds(j * G + t * 16, 16)]
      dst16 = dstrow[pl.ds(t * 16, 16)]
      ex_h = []
      for h in range(H):
        av = plsc.load_gather(asd_v, [src16 * 8 + h])
        bv = plsc.load_gather(asd_v, [dst16 * 8 + (H + h)])
        al = av + bv
        al = jnp.where(al >= 0, al, 0.2 * al)
        ex_h.append(jnp.exp(al))
      exs.append(ex_h)

    gat.wait()

    for t in range(G // 16):
      for l in range(16):
        e = t * 16 + l
        wv = [jnp.full((16,), exs[t][h][l]) for h in range(H)]
        aug = zero16
        for h in range(H):
          aug = jnp.where(head_masks[h], wv[h], aug)
        for k in range(HC // 16):
          stag[e, pl.ds(k * 16, 16)] = rows[e, pl.ds(k * 16, 16)] * wv[k // 2]
        stag[e, pl.ds(HC, 16)] = aug

    pltpu.sync_copy(stag, agg_sh.at[dst2d_v.at[j]], add=True)
    return carry

  lax.fori_loop(0, GPT, group, 0)
  plsc.subcore_barrier()
  pltpu.sync_copy(agg_sh.at[pl.ds(s * NPC, NPC)],
                  parts_hbm.at[c, pl.ds(s * NPC, NPC)])


def _run_edges(srcf, dst2d, asd_flat, xh, zeros):
  mesh = plsc.VectorSubcoreMesh(core_axis_name="c", subcore_axis_name="s")
  fn = pl.kernel(
      _edge_kernel,
      out_type=jax.ShapeDtypeStruct((NCORES, N, AUG), _f32),
      mesh=mesh,
      scratch_types=[
          pltpu.VMEM((8 * N,), _f32),
          pltpu.VMEM((EPT,), jnp.int32),
          pltpu.VMEM((GPT, G), jnp.int32),
          pltpu.VMEM((G,), jnp.int32),
          pltpu.VMEM((G, HC), _f32),
          pltpu.VMEM((G, AUG), _f32),
          pltpu.VMEM_SHARED((N, AUG), _f32),
          pltpu.SemaphoreType.DMA,
      ],
  )
  return fn(srcf, dst2d, asd_flat, xh, zeros)


def _head_kernel(parts_ref, batch_ref, b1_ref, r_ref, wd1_ref, bd1_ref,
                 wd2_ref, bd2_ref, out_ref, gmax):
  i = pl.program_id(0)

  @pl.when(i == 0)
  def _():
    gmax[...] = jnp.full((NGRAPH, C), -1e30, _f32)

  p0 = parts_ref[0]
  p1 = parts_ref[1]
  agg = p0[:, :HC] + p1[:, :HC]
  den = p0[:, HC:HC + H] + p1[:, HC:HC + H]
  den_rep = jnp.dot(den, r_ref[...], preferred_element_type=_f32)
  h1 = agg / (den_rep + 1e-16) + b1_ref[...]
  h1 = jnp.where(h1 >= 0, h1, 0.01 * h1)
  h2 = jnp.dot(h1, wd1_ref[...], preferred_element_type=_f32) + bd1_ref[...]
  h2 = jnp.where(h2 >= 0, h2, 0.01 * h2)

  b = batch_ref[...]
  for g in range(NGRAPH):
    sel = jnp.where(b == g, h2, -1e30)
    m = jnp.max(sel, axis=0, keepdims=True)
    gmax[g:g + 1, :] = jnp.maximum(gmax[g:g + 1, :], m)

  @pl.when(i == pl.num_programs(0) - 1)
  def _():
    gf = gmax[...]
    gf = jnp.where(gf > -1e29, gf, 0.0)
    out_ref[...] = jnp.dot(gf, wd2_ref[...], preferred_element_type=_f32) \
        + bd2_ref[...]


def _run_head(parts, batch2d, b1, R, Wd1, bd1, Wd2, bd2):
  bn = 1000
  return pl.pallas_call(
      _head_kernel,
      grid=(N // bn,),
      in_specs=[
          pl.BlockSpec((NCORES, bn, AUG), lambda i: (0, i, 0)),
          pl.BlockSpec((bn, 1), lambda i: (i, 0)),
          pl.BlockSpec((1, HC), lambda i: (0, 0)),
          pl.BlockSpec((H, HC), lambda i: (0, 0)),
          pl.BlockSpec((HC, C), lambda i: (0, 0)),
          pl.BlockSpec((1, C), lambda i: (0, 0)),
          pl.BlockSpec((C, 1), lambda i: (0, 0)),
          pl.BlockSpec((1, 1), lambda i: (0, 0)),
      ],
      out_specs=pl.BlockSpec((NGRAPH, 1), lambda i: (0, 0)),
      out_shape=jax.ShapeDtypeStruct((NGRAPH, 1), _f32),
      scratch_shapes=[pltpu.VMEM((NGRAPH, C), _f32)],
  )(parts, batch2d, b1, R, Wd1, bd1, Wd2, bd2)


_RIDX = np.arange(HC)
_HIDX = _RIDX // C
_R_NP = np.zeros((H, HC), np.float32)
for _h in range(H):
  _R_NP[_h, _h * C:(_h + 1) * C] = 1.0


def kernel(x, edge_index, edge_attr, batch, W1, att_src, att_dst, b1, Wd1,
           bd1, Wd2, bd2):
  del edge_attr
  src = edge_index[0]
  dst2d = edge_index[1].reshape(E // G, G)

  P = jnp.zeros((HC, 2 * H), _f32)
  P = P.at[_RIDX, _HIDX].set(att_src.reshape(HC))
  P = P.at[_RIDX, H + _HIDX].set(att_dst.reshape(HC))
  R = jnp.asarray(_R_NP)

  xh, asd = _run_proj(x, W1, P)
  zeros = jnp.zeros((NPC, AUG), _f32)
  parts = _run_edges(src, dst2d, asd.reshape(8 * N), xh, zeros)
  out = _run_head(parts, batch.reshape(N, 1), b1.reshape(1, HC), R, Wd1,
                  bd1.reshape(1, C), Wd2, bd2.reshape(1, 1))
  return out


# speedup vs baseline: 42.7353x; 42.7353x over previous
import jax
import jax.numpy as jnp
import numpy as np
from jax import lax
from jax.experimental import pallas as pl
from jax.experimental.pallas import tpu as pltpu
from jax.experimental.pallas import tpu_sc as plsc

N = 10000
E = 320000
FIN = 128
H = 4
C = 32
HC = H * C
NGRAPH = 64

NCORES = 2
NSUB = 16
EPT = E // NSUB
G = 80
GPT = EPT // G
SCH = 5
NCHUNK = GPT // SCH
NPAD = 10240
HALF = NPAD // NCORES
APC = HALF // NSUB
DPACK = 4
DROWS = HALF // DPACK
DPC = DROWS // NSUB
G2 = G // 2

_f32 = jnp.float32


def _proj_kernel(x_ref, w1_ref, p_ref, xh_ref, asd_ref):
  xh = jnp.dot(x_ref[...], w1_ref[...], preferred_element_type=_f32)
  xh_ref[...] = xh
  asd_ref[...] = jnp.dot(xh, p_ref[...], preferred_element_type=_f32)


def _run_proj(x, W1, P):
  bn = 1000
  return pl.pallas_call(
      _proj_kernel,
      grid=(N // bn,),
      in_specs=[
          pl.BlockSpec((bn, FIN), lambda i: (i, 0)),
          pl.BlockSpec((FIN, HC), lambda i: (0, 0)),
          pl.BlockSpec((HC, 2 * H), lambda i: (0, 0)),
      ],
      out_specs=[
          pl.BlockSpec((bn, HC), lambda i: (i, 0)),
          pl.BlockSpec((bn, 2 * H), lambda i: (i, 0)),
      ],
      out_shape=[
          jax.ShapeDtypeStruct((N, HC), _f32),
          jax.ShapeDtypeStruct((N, 2 * H), _f32),
      ],
  )(x, W1, P)


def _edge_kernel(srcf_hbm, dst4d_hbm, asrc_hbm, adst_hbm, xh_hbm, zeros_hbm,
                 parts_hbm, denp_hbm,
                 asrc_v, adst_v, srcf_v, dst2d_v, ldst_v, didx_v, rows, stag2,
                 agg_sh, den_sh, sem):
  c = lax.axis_index("c")
  s = lax.axis_index("s")
  lo = c * HALF

  pltpu.sync_copy(asrc_hbm, asrc_v)
  pltpu.sync_copy(adst_hbm.at[c], adst_v)
  pltpu.sync_copy(zeros_hbm, agg_sh.at[pl.ds(s * APC, APC)])
  pltpu.sync_copy(zeros_hbm.at[pl.ds(0, DPC)], den_sh.at[pl.ds(s * DPC, DPC)])
  pltpu.sync_copy(zeros_hbm.at[pl.ds(0, 16)], stag2)
  plsc.subcore_barrier()

  iota16 = lax.iota(jnp.int32, 16)

  def group(j, carry):
    jc = j % SCH

    @pl.when(jc == 0)
    def _():
      pltpu.sync_copy(
          srcf_hbm.at[pl.ds(s * EPT + (j // SCH) * (SCH * G), SCH * G)],
          srcf_v)
      pltpu.sync_copy(dst4d_hbm.at[s, j // SCH], dst2d_v)

    gat = pltpu.async_copy(xh_hbm.at[srcf_v.at[pl.ds(jc * G, G)]], rows, sem)

    exs = []
    offs = []
    for t in range(G // 16):
      src16 = srcf_v[pl.ds(jc * G + t * 16, 16)]
      dst16 = dst2d_v[jc, pl.ds(t * 16, 16)]
      inh = (dst16 >= lo) & (dst16 < lo + HALF)
      ldst16 = jnp.where(inh, dst16 - lo, jnp.bitwise_and(dst16, 4095))
      ldst_v[0, pl.ds(t * 16, 16)] = ldst16
      didx_v[0, pl.ds(t * 16, 16)] = lax.shift_right_logical(ldst16, 2)
      offs.append(lax.shift_left(jnp.bitwise_and(ldst16, 3), 2))
      ex_h = []
      for h in range(H):
        av = plsc.load_gather(asrc_v, [src16 * 4 + h])
        bv = plsc.load_gather(adst_v, [ldst16 * 4 + h])
        al = av + bv
        al = jnp.where(al >= 0, al, 0.2 * al)
        ex = jnp.exp(al)
        ex_h.append(jnp.where(inh, ex, 0.0))
      exs.append(ex_h)

    gat.wait()

    for t in range(G // 16):
      for l in range(16):
        e = t * 16 + l
        wv = [jnp.full((16,), exs[t][h][l]) for h in range(H)]
        off = offs[t][l]
        for k in range(HC // 16):
          rows[e, pl.ds(k * 16, 16)] = \
              rows[e, pl.ds(k * 16, 16)] * wv[k // 2]
        aug = jnp.zeros((16,), _f32)
        for h in range(H):
          aug = jnp.where(iota16 == off + h, wv[h], aug)
        stag2[l, pl.ds(0, 16)] = aug
      pltpu.sync_copy(
          stag2, den_sh.at[didx_v.at[0, pl.ds(t * 16, 16)]], add=True)
    pltpu.sync_copy(rows, agg_sh.at[ldst_v.at[0]], add=True)
    return carry

  lax.fori_loop(0, GPT, group, 0)
  plsc.subcore_barrier()
  pltpu.sync_copy(agg_sh.at[pl.ds(s * APC, APC)],
                  parts_hbm.at[c, pl.ds(s * APC, APC)])
  pltpu.sync_copy(den_sh.at[pl.ds(s * DPC, DPC)],
                  denp_hbm.at[c, pl.ds(s * DPC, DPC)])


def _run_edges(srcf, dst4d, asrc, adst, xh, zeros):
  mesh = plsc.VectorSubcoreMesh(core_axis_name="c", subcore_axis_name="s")
  fn = pl.kernel(
      _edge_kernel,
      out_type=[
          jax.ShapeDtypeStruct((NCORES, HALF, HC), _f32),
          jax.ShapeDtypeStruct((NCORES, DROWS, HC), _f32),
      ],
      mesh=mesh,
      compiler_params=pltpu.CompilerParams(needs_layout_passes=False),
      scratch_types=[
          pltpu.VMEM((4 * N,), _f32),
          pltpu.VMEM((4 * HALF,), _f32),
          pltpu.VMEM((SCH * G,), jnp.int32),
          pltpu.VMEM((SCH, G), jnp.int32),
          pltpu.VMEM((1, G), jnp.int32),
          pltpu.VMEM((1, G), jnp.int32),
          pltpu.VMEM((G, HC), _f32),
          pltpu.VMEM((16, HC), _f32),
          pltpu.VMEM_SHARED((HALF, HC), _f32),
          pltpu.VMEM_SHARED((DROWS, HC), _f32),
          pltpu.SemaphoreType.DMA,
      ],
  )
  return fn(srcf, dst4d, asrc, adst, xh, zeros)


def _head_kernel(parts_ref, denp_ref, batch_ref, b1_ref, wd1_ref, bd1_ref,
                 wd2_ref, bd2_ref, out_ref, gmax):
  i = pl.program_id(0)

  @pl.when(i == 0)
  def _():
    gmax[...] = jnp.full((NGRAPH, C), -1e30, _f32)

  agg = parts_ref[0]
  dpk = denp_ref[0]

  per_q = []
  for q in range(DPACK):
    cols = [jnp.broadcast_to(dpk[:, q * H + h:q * H + h + 1], (256, C))
            for h in range(H)]
    per_q.append(jnp.concatenate(cols, axis=1).reshape(256, 1, HC))
  den_rep = jnp.concatenate(per_q, axis=1).reshape(256 * DPACK, HC)

  h1 = agg / (den_rep + 1e-16) + b1_ref[...]
  h1 = jnp.where(h1 >= 0, h1, 0.01 * h1)
  h2 = jnp.dot(h1, wd1_ref[...], preferred_element_type=_f32) + bd1_ref[...]
  h2 = jnp.where(h2 >= 0, h2, 0.01 * h2)

  b = batch_ref[...]
  for g in range(NGRAPH):
    sel = jnp.where(b == g, h2, -1e30)
    m = jnp.max(sel, axis=0, keepdims=True)
    gmax[g:g + 1, :] = jnp.maximum(gmax[g:g + 1, :], m)

  @pl.when(i == pl.num_programs(0) - 1)
  def _():
    gf = gmax[...]
    gf = jnp.where(gf > -1e29, gf, 0.0)
    out_ref[...] = jnp.dot(gf, wd2_ref[...], preferred_element_type=_f32) \
        + bd2_ref[...]


def _run_head(parts, denp, batchp, b1, Wd1, bd1, Wd2, bd2):
  bn = 1024
  nbh = HALF // bn
  return pl.pallas_call(
      _head_kernel,
      grid=(NPAD // bn,),
      in_specs=[
          pl.BlockSpec((1, bn, HC), lambda i: (i // nbh, i % nbh, 0)),
          pl.BlockSpec((1, bn // DPACK, HC), lambda i: (i // nbh, i % nbh, 0)),
          pl.BlockSpec((bn, 1), lambda i: (i, 0)),
          pl.BlockSpec((1, HC), lambda i: (0, 0)),
          pl.BlockSpec((HC, C), lambda i: (0, 0)),
          pl.BlockSpec((1, C), lambda i: (0, 0)),
          pl.BlockSpec((C, 1), lambda i: (0, 0)),
          pl.BlockSpec((1, 1), lambda i: (0, 0)),
      ],
      out_specs=pl.BlockSpec((NGRAPH, 1), lambda i: (0, 0)),
      out_shape=jax.ShapeDtypeStruct((NGRAPH, 1), _f32),
      scratch_shapes=[pltpu.VMEM((NGRAPH, C), _f32)],
  )(parts, denp, batchp, b1, Wd1, bd1, Wd2, bd2)


_RIDX = np.arange(HC)
_HIDX = _RIDX // C


def kernel(x, edge_index, edge_attr, batch, W1, att_src, att_dst, b1, Wd1,
           bd1, Wd2, bd2):
  del edge_attr
  src = edge_index[0]
  dst4d = edge_index[1].reshape(NSUB, NCHUNK, SCH, G)

  P = jnp.zeros((HC, 2 * H), _f32)
  P = P.at[_RIDX, _HIDX].set(att_src.reshape(HC))
  P = P.at[_RIDX, H + _HIDX].set(att_dst.reshape(HC))

  xh, asd = _run_proj(x, W1, P)
  asrc = asd[:, :H].reshape(H * N)
  adst = jnp.concatenate(
      [asd[:, H:], jnp.zeros((NPAD - N, H), _f32)]).reshape(NCORES, H * HALF)
  zeros = jnp.zeros((APC, HC), _f32)
  parts, denp = _run_edges(src, dst4d, asrc, adst, xh, zeros)
  batchp = jnp.concatenate(
      [batch, jnp.full((NPAD - N,), NGRAPH, jnp.int32)]).reshape(NPAD, 1)
  out = _run_head(parts, denp, batchp, b1.reshape(1, HC), Wd1,
                  bd1.reshape(1, C), Wd2, bd2.reshape(1, 1))
  return out
